# baseline (speedup 1.0000x reference)
import functools

import jax
import jax.numpy as jnp
from jax import lax
from jax.experimental import pallas as pl
from jax.experimental.pallas import tpu as pltpu
from jax.experimental.pallas import tpu_sc as plsc

BATCH = 16384
CLASSES = 1000
GAMMA = 2.0
BLK = 1024
NB = BATCH // BLK

NWORK = 32
CH = BATCH // NWORK


def _sc_alpha_gather(alpha1d, targets):
    mesh = plsc.VectorSubcoreMesh(core_axis_name="c", subcore_axis_name="s")

    @functools.partial(
        pl.kernel,
        mesh=mesh,
        out_type=jax.ShapeDtypeStruct((BATCH,), jnp.float32),
        scratch_types=[
            pltpu.VMEM((CH // 128, 128), jnp.int32),
            pltpu.VMEM((CH,), jnp.float32),
            pltpu.SemaphoreType.DMA,
        ],
    )
    def gather_kernel(alpha_hbm, t_hbm, out_hbm, t_v, at_v, sem):
        wid = lax.axis_index("s") * 2 + lax.axis_index("c")
        base = wid * CH
        for k in range(CH // 128):
            pltpu.sync_copy(t_hbm.at[pl.ds(base + 128 * k, 128)], t_v.at[k])
        for k in range(CH // 128):
            pltpu.async_copy(
                alpha_hbm.at[t_v.at[k]], at_v.at[pl.ds(128 * k, 128)], sem
            ).wait()
        pltpu.sync_copy(at_v, out_hbm.at[pl.ds(base, CH)])

    return gather_kernel(alpha1d, targets)


def _focal_body(x_ref, t_ref, at_ref, out_ref):
    i = pl.program_id(0)
    x = x_ref[...]
    t = t_ref[0, 0, :]
    m = jnp.max(x, axis=0, keepdims=True)
    e = jnp.exp(x - m)
    s = jnp.sum(e, axis=0, keepdims=True)

    row = jax.lax.broadcasted_iota(jnp.int32, (CLASSES, BLK), 0)
    mask = row == t[None, :]
    xt = jnp.sum(jnp.where(mask, x, 0.0), axis=0, keepdims=True)
    at = at_ref[0, :, :]

    log_p = (xt - m) - jnp.log(s)
    p = jnp.exp(log_p)
    omp = 1.0 - p
    loss = -at * (omp * omp) * log_p
    part = jnp.sum(loss)

    @pl.when(i == 0)
    def _():
        out_ref[0, 0] = 0.0

    out_ref[0, 0] += part

    @pl.when(i == NB - 1)
    def _():
        out_ref[0, 0] = out_ref[0, 0] * (1.0 / BATCH)


def kernel(inputs, targets, alpha):
    at1d = _sc_alpha_gather(alpha.reshape(CLASSES), targets)
    xT = inputs.T
    t3 = targets.reshape(NB, 1, BLK)
    at3 = at1d.reshape(NB, 1, BLK)
    out = pl.pallas_call(
        _focal_body,
        grid=(NB,),
        in_specs=[
            pl.BlockSpec((CLASSES, BLK), lambda i: (0, i)),
            pl.BlockSpec((1, 1, BLK), lambda i: (i, 0, 0)),
            pl.BlockSpec((1, 1, BLK), lambda i: (i, 0, 0)),
        ],
        out_specs=pl.BlockSpec(memory_space=pltpu.SMEM),
        out_shape=jax.ShapeDtypeStruct((1, 1), jnp.float32),
    )(xT, t3, at3)
    return out[0, 0]


# speedup vs baseline: 1.8723x; 1.8723x over previous
import jax
import jax.numpy as jnp
from jax.experimental import pallas as pl
from jax.experimental.pallas import tpu as pltpu

BATCH = 16384
CLASSES = 1000
GAMMA = 2.0
BLK = 1024
NB = BATCH // BLK


def _focal_body(x_ref, t_ref, a_ref, out_ref):
    i = pl.program_id(0)
    x = x_ref[...]
    t = t_ref[0, 0, :]
    m = jnp.max(x, axis=0, keepdims=True)
    e = jnp.exp(x - m)
    s = jnp.sum(e, axis=0, keepdims=True)

    row = jax.lax.broadcasted_iota(jnp.int32, (CLASSES, BLK), 0)
    onehot = (row == t[None, :]).astype(jnp.float32)
    xt = jnp.sum(x * onehot, axis=0, keepdims=True)
    a = a_ref[...]
    at = jax.lax.dot_general(
        a, onehot, (((0,), (0,)), ((), ())),
        preferred_element_type=jnp.float32,
    )

    log_p = (xt - m) - jnp.log(s)
    p = jnp.exp(log_p)
    omp = 1.0 - p
    loss = -at * (omp * omp) * log_p
    part = jnp.sum(loss)

    @pl.when(i == 0)
    def _():
        out_ref[0, 0] = 0.0

    out_ref[0, 0] += part

    @pl.when(i == NB - 1)
    def _():
        out_ref[0, 0] = out_ref[0, 0] * (1.0 / BATCH)


def kernel(inputs, targets, alpha):
    xT = inputs.T
    t3 = targets.reshape(NB, 1, BLK)
    out = pl.pallas_call(
        _focal_body,
        grid=(NB,),
        in_specs=[
            pl.BlockSpec((CLASSES, BLK), lambda i: (0, i)),
            pl.BlockSpec((1, 1, BLK), lambda i: (i, 0, 0)),
            pl.BlockSpec((CLASSES, 1), lambda i: (0, 0)),
        ],
        out_specs=pl.BlockSpec(memory_space=pltpu.SMEM),
        out_shape=jax.ShapeDtypeStruct((1, 1), jnp.float32),
    )(xT, t3, alpha)
    return out[0, 0]


# speedup vs baseline: 1.9268x; 1.0291x over previous
import jax
import jax.numpy as jnp
from jax.experimental import pallas as pl
from jax.experimental.pallas import tpu as pltpu

BATCH = 16384
CLASSES = 1000
GAMMA = 2.0
BLK = 1024
NB = BATCH // BLK


def _focal_body(x_ref, t_ref, a_ref, out_ref):
    i = pl.program_id(0)
    x = x_ref[...]
    t = t_ref[0, 0, :]
    m = jnp.max(x, axis=0, keepdims=True)
    e = jnp.exp(x - m)
    s = jnp.sum(e, axis=0, keepdims=True)

    row = jax.lax.broadcasted_iota(jnp.int32, (CLASSES, BLK), 0)
    mask = row == t[None, :]
    xt = jnp.sum(jnp.where(mask, x, 0.0), axis=0, keepdims=True)
    onehot = mask.astype(jnp.bfloat16)
    a = a_ref[...].astype(jnp.bfloat16)
    at = jax.lax.dot_general(
        a, onehot, (((0,), (0,)), ((), ())),
        preferred_element_type=jnp.float32,
    )

    log_p = (xt - m) - jnp.log(s)
    p = jnp.exp(log_p)
    omp = 1.0 - p
    loss = -at * (omp * omp) * log_p
    part = jnp.sum(loss)

    @pl.when(i == 0)
    def _():
        out_ref[0, 0] = 0.0

    out_ref[0, 0] += part

    @pl.when(i == NB - 1)
    def _():
        out_ref[0, 0] = out_ref[0, 0] * (1.0 / BATCH)


def kernel(inputs, targets, alpha):
    xT = inputs.T
    t3 = targets.reshape(NB, 1, BLK)
    out = pl.pallas_call(
        _focal_body,
        grid=(NB,),
        in_specs=[
            pl.BlockSpec((CLASSES, BLK), lambda i: (0, i)),
            pl.BlockSpec((1, 1, BLK), lambda i: (i, 0, 0)),
            pl.BlockSpec((CLASSES, 1), lambda i: (0, 0)),
        ],
        out_specs=pl.BlockSpec(memory_space=pltpu.SMEM),
        out_shape=jax.ShapeDtypeStruct((1, 1), jnp.float32),
    )(xT, t3, alpha)
    return out[0, 0]
